# Initial kernel scaffold; baseline (speedup 1.0000x reference)
import jax
import jax.numpy as jnp
from jax.experimental import pallas as pl


def kernel(input_tokens, start, weight):
    raise NotImplementedError("write your pallas kernel here")


# speedup vs baseline: 2.2326x; 2.2326x over previous
import jax
import jax.numpy as jnp
from jax import lax
from jax.experimental import pallas as pl
from jax.experimental.pallas import tpu as pltpu
from jax.experimental.pallas import tpu_sc as plsc

PAD = 1
B, T, D = 4, 8192, 1024
NC, NS, L = 2, 16, 16
NW = NC * NS
SEG = (B * T) // NW
SEGS_PER_ROW = T // SEG
CHUNK = 32
NCHUNK = SEG // CHUNK


def _sc_body(tok_hbm, start_hbm, weight_hbm, out_hbm,
             tokbuf, idxbuf, startbuf, buf0, buf1, buf2,
             gs0, gs1, gs2, ps0, ps1, ps2):
    cid = lax.axis_index("c")
    sid = lax.axis_index("s")
    wid = sid * NC + cid
    r = wid // SEGS_PER_ROW
    s = wid % SEGS_PER_ROW
    base = pl.multiple_of(s * SEG, SEG)

    pltpu.sync_copy(start_hbm, startbuf)
    pltpu.sync_copy(tok_hbm.at[r], tokbuf)
    sv = startbuf[...]

    def pre_body(j, acc):
        v = tokbuf[pl.ds(pl.multiple_of(j * L, L), L)]
        return acc + jnp.sum((v != PAD).astype(jnp.int32))

    pre = lax.fori_loop(0, s * (SEG // L), pre_body, jnp.int32(0))

    def loc_body(j, carry):
        v = tokbuf[pl.ds(pl.multiple_of(base + j * L, L), L)]
        m = (v != PAD).astype(jnp.int32)
        c = plsc.cumsum(m)
        idxbuf[pl.ds(pl.multiple_of(j * L, L), L)] = (c + carry) * m + sv
        return carry + jnp.sum(m)

    lax.fori_loop(0, SEG // L, loc_body, pre)

    bufs = (buf0, buf1, buf2)
    gsems = (gs0, gs1, gs2)
    psems = (ps0, ps1, ps2)
    gcp, pcp = {}, {}

    def gstart(k):
        p = k % 3
        cp = pltpu.make_async_copy(
            weight_hbm.at[idxbuf.at[pl.ds(k * CHUNK, CHUNK)]], bufs[p], gsems[p])
        cp.start()
        gcp[k] = cp

    def pstart(k):
        p = k % 3
        cp = pltpu.make_async_copy(
            bufs[p],
            out_hbm.at[r, pl.ds(pl.multiple_of(base + k * CHUNK, CHUNK), CHUNK)],
            psems[p])
        cp.start()
        pcp[k] = cp

    gstart(0)
    gstart(1)
    for k in range(NCHUNK):
        gcp[k].wait()
        pstart(k)
        nk = k + 2
        if nk < NCHUNK:
            if nk >= 3:
                pcp[nk - 3].wait()
            gstart(nk)
    for k in range(NCHUNK - 3, NCHUNK):
        pcp[k].wait()


_mesh = plsc.VectorSubcoreMesh(core_axis_name="c", subcore_axis_name="s",
                               num_cores=NC, num_subcores=NS)

_sc_call = pl.kernel(
    _sc_body,
    out_type=jax.ShapeDtypeStruct((B, T, D), jnp.float32),
    mesh=_mesh,
    scratch_types=[
        pltpu.VMEM((T,), jnp.int32),
        pltpu.VMEM((SEG,), jnp.int32),
        pltpu.VMEM((L,), jnp.int32),
        pltpu.VMEM((CHUNK, D), jnp.float32),
        pltpu.VMEM((CHUNK, D), jnp.float32),
        pltpu.VMEM((CHUNK, D), jnp.float32),
        pltpu.SemaphoreType.DMA,
        pltpu.SemaphoreType.DMA,
        pltpu.SemaphoreType.DMA,
        pltpu.SemaphoreType.DMA,
        pltpu.SemaphoreType.DMA,
        pltpu.SemaphoreType.DMA,
    ],
    name="sinusoidal_pos_emb_lookup",
    compiler_params=pltpu.CompilerParams(needs_layout_passes=False),
)


def kernel(input_tokens, start, weight):
    if start is None:
        start = 0
    start_vec = jnp.full((L,), start, dtype=jnp.int32)
    return _sc_call(input_tokens.astype(jnp.int32), start_vec,
                    weight.astype(jnp.float32))
